# baseline (speedup 1.0000x reference)
import jax
import jax.numpy as jnp
from jax import lax
from jax.experimental import pallas as pl

NUM_CLASSES = 80
N = 134400
BLOCK = 8960
BR = BLOCK // 32


def _roll(v, k):
    return jnp.roll(v, -k, axis=1)


def _criterion_block(cls_ref, p_ref, g_ref, a_ref, r_ref, lab_ref,
                     cls_out, reg_out, box_out, npos_out):
    i = pl.program_id(0)

    @pl.when(i == 0)
    def _init():
        cls_out[...] = jnp.zeros_like(cls_out)
        reg_out[...] = jnp.zeros_like(reg_out)
        box_out[...] = jnp.zeros_like(box_out)
        npos_out[...] = jnp.zeros_like(npos_out)

    lrow = lab_ref[...]
    labels = lrow.reshape(BLOCK, 1).astype(jnp.int32)
    posb = (labels >= 0) & (labels < NUM_CLASSES)
    x = cls_ref[...]
    col = lax.broadcasted_iota(jnp.int32, x.shape, 1)
    m = (col == labels) & posb
    mf = m.astype(jnp.float32)
    e = jnp.exp(-jnp.abs(x))
    d = 1.0 + e
    r = 1.0 / d
    er = e * r
    ell = jnp.log(d)
    xneg = x < 0.0
    sg = jnp.where(m ^ xneg, er, r)
    sp = jnp.maximum(x, 0.0) - x * mf + ell
    alpha_t = 0.75 - 0.5 * mf
    cls_sum = jnp.sum(alpha_t * sg * sg * sp)

    P = p_ref[...]
    G = g_ref[...]
    A = a_ref[...]
    R = r_ref[...]

    lane = lax.broadcasted_iota(jnp.int32, P.shape, 1)
    coord = lane & 3
    posgf = posb.reshape(BR, 32).astype(jnp.float32)
    expand = (lax.broadcasted_iota(jnp.int32, (32, 128), 1) // 4
              == lax.broadcasted_iota(jnp.int32, (32, 128), 0)
              ).astype(jnp.float32)
    pos128f = jax.lax.dot_general(
        posgf, expand, (((1,), (0,)), ((), ())),
        preferred_element_type=jnp.float32)

    M = jnp.minimum(P, G)
    X = jnp.maximum(P, G)
    iwih = jnp.clip(_roll(M, 2) - X, 0.0)
    inter = iwih * _roll(iwih, 1)
    dP = jnp.clip(_roll(P, 2) - P, 0.0)
    a1 = dP * _roll(dP, 1)
    dGr = _roll(G, 2) - G
    dG = jnp.clip(dGr, 0.0)
    a2 = dG * _roll(dG, 1)
    union = a1 + a2 - inter
    iou = inter / jnp.clip(union, 1e-7)
    cwch = jnp.clip(_roll(X, 2) - M, 0.0)
    ac = cwch * _roll(cwch, 1)
    giou = iou - (ac - union) / jnp.clip(ac, 1e-7)
    c0 = (coord == 0).astype(jnp.float32) * pos128f
    reg_sum = jnp.sum((1.0 - giou) * c0)
    npos_sum = jnp.sum(c0)

    S = (G + _roll(G, 2)) * 0.5
    A2 = _roll(A, 2)
    cxy = (S - A) / A2
    gwh = jnp.clip(dGr, 1e-7)
    wh = jnp.log(gwh / A2)
    t1 = jnp.abs(R - cxy)
    t2 = jnp.abs(_roll(R, 2) - wh)
    c01 = (coord < 2).astype(jnp.float32) * pos128f
    box_sum = jnp.sum((t1 + t2) * c01)

    cls_out[...] += cls_sum
    reg_out[...] += reg_sum
    box_out[...] += box_sum
    npos_out[...] += npos_sum


@jax.jit
def kernel(pred_cls, pred_reg, pred_box, gt_box, anchors, tgt_labels):
    lab_f = tgt_labels.astype(jnp.float32).reshape(1, N)
    P = pred_box.reshape(N // 32, 128)
    G = gt_box.reshape(N // 32, 128)
    A = anchors.reshape(N // 32, 128)
    R = pred_reg.reshape(N // 32, 128)
    grid = (N // BLOCK,)
    scalar_spec = pl.BlockSpec((1, 1), lambda i: (0, 0))
    out = pl.pallas_call(
        _criterion_block,
        grid=grid,
        in_specs=[
            pl.BlockSpec((BLOCK, NUM_CLASSES), lambda i: (i, 0)),
            pl.BlockSpec((BR, 128), lambda i: (i, 0)),
            pl.BlockSpec((BR, 128), lambda i: (i, 0)),
            pl.BlockSpec((BR, 128), lambda i: (i, 0)),
            pl.BlockSpec((BR, 128), lambda i: (i, 0)),
            pl.BlockSpec((1, BLOCK), lambda i: (0, i)),
        ],
        out_specs=(scalar_spec, scalar_spec, scalar_spec, scalar_spec),
        out_shape=tuple(jax.ShapeDtypeStruct((1, 1), jnp.float32)
                        for _ in range(4)),
    )(pred_cls, P, G, A, R, lab_f)
    cls_sum, reg_sum, box_sum, npos = (o[0, 0] for o in out)
    num_fgs = jnp.maximum(npos, 1.0)
    return jnp.stack([cls_sum, reg_sum, box_sum]) / num_fgs


# speedup vs baseline: 2.8964x; 2.8964x over previous
import jax
import jax.numpy as jnp
from jax import lax
from jax.experimental import pallas as pl

NUM_CLASSES = 80
N = 134400
BLOCK = 8960


def _criterion_block(pred_cls_ref, rows_ref, cls_ref, clsm_ref, reg_ref,
                     box_ref, npos_ref):
    i = pl.program_id(0)

    @pl.when(i == 0)
    def _init():
        cls_ref[...] = jnp.zeros_like(cls_ref)
        clsm_ref[...] = jnp.zeros_like(clsm_ref)
        reg_ref[...] = jnp.zeros_like(reg_ref)
        box_ref[...] = jnp.zeros_like(box_ref)
        npos_ref[...] = jnp.zeros_like(npos_ref)

    rows = rows_ref[...]
    prx, pry, prw, prh = (rows[0:1], rows[1:2], rows[2:3], rows[3:4])
    px1, py1, px2, py2 = (rows[4:5], rows[5:6], rows[6:7], rows[7:8])
    gx1, gy1, gx2, gy2 = (rows[8:9], rows[9:10], rows[10:11], rows[11:12])
    ax, ay, aw, ah = (rows[12:13], rows[13:14], rows[14:15], rows[15:16])
    lrow = rows[16:17]
    pos_row = (lrow >= 0.0) & (lrow < float(NUM_CLASSES))
    pos_f = pos_row.astype(jnp.float32)

    labels = lrow.reshape(BLOCK, 1).astype(jnp.int32)
    posb = (labels >= 0) & (labels < NUM_CLASSES)
    x = pred_cls_ref[...]
    col = lax.broadcasted_iota(jnp.int32, x.shape, 1)
    m = (col == labels) & posb
    mf = m.astype(jnp.float32)
    e = jnp.exp(-jnp.abs(x))
    d = 1.0 + e
    r = 1.0 / d
    er = e * r
    ell = jnp.log(d)
    xneg = x < 0.0
    sg = jnp.where(m ^ xneg, er, r)
    sp = jnp.maximum(x, 0.0) - x * mf + ell
    g = sg * sg * sp
    cls_sum = jnp.sum(g)
    clsm_sum = jnp.sum(jnp.where(m, g, 0.0))

    iw = jnp.clip(jnp.minimum(px2, gx2) - jnp.maximum(px1, gx1), 0.0)
    ih = jnp.clip(jnp.minimum(py2, gy2) - jnp.maximum(py1, gy1), 0.0)
    inter = iw * ih
    a1 = jnp.clip(px2 - px1, 0.0) * jnp.clip(py2 - py1, 0.0)
    a2 = jnp.clip(gx2 - gx1, 0.0) * jnp.clip(gy2 - gy1, 0.0)
    union = a1 + a2 - inter
    iou = inter / jnp.clip(union, 1e-7)
    cw = jnp.maximum(px2, gx2) - jnp.minimum(px1, gx1)
    ch = jnp.maximum(py2, gy2) - jnp.minimum(py1, gy1)
    area_c = jnp.clip(cw, 0.0) * jnp.clip(ch, 0.0)
    giou = iou - (area_c - union) / jnp.clip(area_c, 1e-7)
    reg_sum = jnp.sum((1.0 - giou) * pos_f)

    gw = jnp.clip(gx2 - gx1, 1e-7)
    gh = jnp.clip(gy2 - gy1, 1e-7)
    ecx = ((gx1 + gx2) * 0.5 - ax) / aw
    ecy = ((gy1 + gy2) * 0.5 - ay) / ah
    ew = jnp.log(gw / aw)
    eh = jnp.log(gh / ah)
    l1 = (jnp.abs(prx - ecx) + jnp.abs(pry - ecy)
          + jnp.abs(prw - ew) + jnp.abs(prh - eh))
    box_sum = jnp.sum(l1 * pos_f)

    cls_ref[...] += cls_sum
    clsm_ref[...] += clsm_sum
    reg_ref[...] += reg_sum
    box_ref[...] += box_sum
    npos_ref[...] += jnp.sum(pos_f)


@jax.jit
def kernel(pred_cls, pred_reg, pred_box, gt_box, anchors, tgt_labels):
    lab_f = tgt_labels.astype(jnp.float32)[:, None]
    rows = jnp.concatenate(
        [pred_reg, pred_box, gt_box, anchors, lab_f], axis=1).T
    grid = (N // BLOCK,)
    scalar_spec = pl.BlockSpec((1, 1), lambda i: (0, 0))
    out = pl.pallas_call(
        _criterion_block,
        grid=grid,
        in_specs=[
            pl.BlockSpec((BLOCK, NUM_CLASSES), lambda i: (i, 0)),
            pl.BlockSpec((17, BLOCK), lambda i: (0, i)),
        ],
        out_specs=(scalar_spec,) * 5,
        out_shape=tuple(jax.ShapeDtypeStruct((1, 1), jnp.float32)
                        for _ in range(5)),
    )(pred_cls, rows)
    cls_sum, clsm_sum, reg_sum, box_sum, npos = (o[0, 0] for o in out)
    num_fgs = jnp.maximum(npos, 1.0)
    loss_cls = 0.75 * cls_sum - 0.5 * clsm_sum
    return jnp.stack([loss_cls, reg_sum, box_sum]) / num_fgs
